# baseline (speedup 1.0000x reference)
import jax
import jax.numpy as jnp
from jax import lax
from jax.experimental import pallas as pl
from jax.experimental.pallas import tpu as pltpu
from jax.experimental.pallas import tpu_sc as plsc

N = 10000
NP = 10240
E = 160000
EP = 163840
D = 256
DE = 16
HID = 256
HH = 128
SCALE = 1.0 / (HID ** 0.5)

NTILES = 32
TE = EP // NTILES
CH = 64
NCH = TE // CH
RPT = NP // 16
TCB = NP // 8

_MESH = plsc.VectorSubcoreMesh(core_axis_name="c", subcore_axis_name="s")
_SC_PARAMS = pltpu.CompilerParams(use_tc_tiling_on_sc=False,
                                  needs_layout_passes=False)


def _proj_body(h_ref, w_ref, b_ref, we_ref,
               q_ref, k_ref, vlo_ref, vhi_ref, skip_ref, qw_ref):
    big = jnp.dot(h_ref[...], w_ref[...],
                  preferred_element_type=jnp.float32) + b_ref[...]
    q = big[:, 0:HID]
    q_ref[...] = q
    k_ref[...] = big[:, HID:2 * HID]
    vlo_ref[...] = big[:, 2 * HID:2 * HID + HH]
    vhi_ref[...] = big[:, 2 * HID + HH:3 * HID]
    skip_ref[...] = big[:, 3 * HID:4 * HID]
    qw_ref[...] = jnp.dot(q, we_ref[...], preferred_element_type=jnp.float32)


def _proj(h, wcat, bcat, we):
    outs = [
        jax.ShapeDtypeStruct((NP, HID), jnp.float32),
        jax.ShapeDtypeStruct((NP, HID), jnp.float32),
        jax.ShapeDtypeStruct((NP, HH), jnp.float32),
        jax.ShapeDtypeStruct((NP, HH), jnp.float32),
        jax.ShapeDtypeStruct((NP, HID), jnp.float32),
        jax.ShapeDtypeStruct((NP, DE), jnp.float32),
    ]
    return pl.pallas_call(
        _proj_body,
        grid=(NP // TCB,),
        in_specs=[
            pl.BlockSpec((TCB, D), lambda i: (i, 0)),
            pl.BlockSpec((D, 4 * HID), lambda i: (0, 0)),
            pl.BlockSpec((1, 4 * HID), lambda i: (0, 0)),
            pl.BlockSpec((D, DE), lambda i: (0, 0)),
        ],
        out_specs=[
            pl.BlockSpec((TCB, HID), lambda i: (i, 0)),
            pl.BlockSpec((TCB, HID), lambda i: (i, 0)),
            pl.BlockSpec((TCB, HH), lambda i: (i, 0)),
            pl.BlockSpec((TCB, HH), lambda i: (i, 0)),
            pl.BlockSpec((TCB, HID), lambda i: (i, 0)),
            pl.BlockSpec((TCB, DE), lambda i: (i, 0)),
        ],
        out_shape=outs,
    )(h, wcat, bcat, we)


def _pass1_body(q_hbm, k_hbm, qw_hbm, ea_hbm, src2_hbm, dst2_hbm,
                s_out, agg_out,
                idx_src, idx_dst, qbufs, kbufs, qwbufs, eabufs,
                stage0, sbuf, agg_sp, semg, sems0):
    c = lax.axis_index("c")
    s_ax = lax.axis_index("s")
    wid = c * 16 + s_ax
    ebase = wid * TE
    rbase = s_ax * RPT

    pltpu.sync_copy(src2_hbm.at[pl.ds(wid * NCH, NCH)], idx_src)
    pltpu.sync_copy(dst2_hbm.at[pl.ds(wid * NCH, NCH)], idx_dst)

    zf = jnp.zeros((16,), jnp.float32)

    def _zstage(r, _):
        stage0[r, pl.ds(0, 16)] = zf
        stage0[r, pl.ds(16, 16)] = zf
        return 0
    lax.fori_loop(0, CH, _zstage, 0)

    for z in range(RPT // CH):
        pltpu.sync_copy(stage0, agg_sp.at[pl.ds(rbase + z * CH, CH)])
    plsc.subcore_barrier()

    lanes = lax.iota(jnp.int32, 16)

    def _issue(ci, b):
        ci = lax.rem(ci, NCH)
        pltpu.async_copy(q_hbm.at[idx_dst.at[ci]], qbufs.at[b], semg.at[b])
        pltpu.async_copy(k_hbm.at[idx_src.at[ci]], kbufs.at[b], semg.at[b])
        pltpu.async_copy(qw_hbm.at[idx_dst.at[ci]], qwbufs.at[b], semg.at[b])
        pltpu.async_copy(ea_hbm.at[pl.ds(ebase + ci * CH, CH)],
                         eabufs.at[b], semg.at[b])

    def _drain(ci, b):
        ci = lax.rem(ci, NCH)
        pltpu.make_async_copy(q_hbm.at[idx_dst.at[ci]], qbufs.at[b],
                              semg.at[b]).wait()
        pltpu.make_async_copy(k_hbm.at[idx_src.at[ci]], kbufs.at[b],
                              semg.at[b]).wait()
        pltpu.make_async_copy(qw_hbm.at[idx_dst.at[ci]], qwbufs.at[b],
                              semg.at[b]).wait()
        pltpu.make_async_copy(ea_hbm.at[pl.ds(ebase + ci * CH, CH)],
                              eabufs.at[b], semg.at[b]).wait()

    def _wait_sc(ci, b):
        pltpu.make_async_copy(stage0, agg_sp.at[idx_dst.at[ci]],
                              sems0).wait()

    def _slot(ci, b, first):
        qbuf, kbuf = qbufs.at[b], kbufs.at[b]
        qwbuf, eabuf = qwbufs.at[b], eabufs.at[b]
        _issue(ci + 1, 1 - b)
        _drain(ci, b)
        for g in range(CH // 16):
            ids = g * 16 + lanes
            gid = ebase + ci * CH + ids
            mask = gid < E

            def _dot(f, acc):
                col = jnp.full((16,), f, jnp.int32)
                qv = plsc.load_gather(qbuf, [ids, col])
                kv = plsc.load_gather(kbuf, [ids, col])
                return acc + qv * kv
            acc = lax.fori_loop(0, HID, _dot, jnp.zeros((16,), jnp.float32),
                                unroll=8)
            for f in range(DE):
                col = jnp.full((16,), f, jnp.int32)
                acc = acc + (plsc.load_gather(qwbuf, [ids, col])
                             * plsc.load_gather(eabuf, [ids, col]))
            sv = jnp.where(mask, jnp.exp(acc * SCALE), 0.0)
            sbuf[pl.ds(ci * CH + g * 16, 16)] = sv

        @pl.when(jnp.logical_not(first))
        def _():
            _wait_sc(ci, b)

        def _egroup(g2, _):
            wv = sbuf[pl.ds(ci * CH + g2 * 16, 16)]
            for j16 in range(16):
                j = g2 * 16 + j16
                w = wv[j16]
                stage0[j, pl.ds(0, 16)] = eabuf[j, :] * w
                stage0[j, pl.ds(16, 16)] = jnp.full((16,), w, jnp.float32)
            return 0
        lax.fori_loop(0, CH // 16, _egroup, 0)

        pltpu.async_copy(stage0, agg_sp.at[idx_dst.at[ci]],
                         sems0, add=True)

    _issue(0, 0)

    def _pair(ci2, _):
        _slot(2 * ci2, 0, ci2 == 0)
        _slot(2 * ci2 + 1, 1, False)
        return 0
    lax.fori_loop(0, NCH // 2, _pair, 0)

    _drain(NCH, 0)
    _wait_sc(NCH - 1, 0)
    pltpu.sync_copy(sbuf, s_out.at[pl.ds(ebase, TE)])

    plsc.subcore_barrier()
    pltpu.sync_copy(agg_sp.at[pl.ds(rbase, RPT)],
                    agg_out.at[c, pl.ds(rbase, RPT)])


def _pass1(q, k, qw, ea_p, src2, dst2):
    f = pl.kernel(
        _pass1_body,
        out_type=[
            jax.ShapeDtypeStruct((EP,), jnp.float32),
            jax.ShapeDtypeStruct((2, NP, 32), jnp.float32),
        ],
        mesh=_MESH,
        compiler_params=_SC_PARAMS,
        scratch_types=[
            pltpu.VMEM((NCH, CH), jnp.int32),
            pltpu.VMEM((NCH, CH), jnp.int32),
            pltpu.VMEM((2, CH, HID), jnp.float32),
            pltpu.VMEM((2, CH, HID), jnp.float32),
            pltpu.VMEM((2, CH, DE), jnp.float32),
            pltpu.VMEM((2, CH, DE), jnp.float32),
            pltpu.VMEM((CH, 32), jnp.float32),
            pltpu.VMEM((TE,), jnp.float32),
            pltpu.VMEM_SHARED((NP, 32), jnp.float32),
            pltpu.SemaphoreType.DMA((2,)),
            pltpu.SemaphoreType.DMA,
        ],
    )
    return f(q, k, qw, ea_p, src2, dst2)


def _pass2_body(vh_hbm, s_hbm, src2_hbm, dst2_hbm,
                acc_out,
                idx_src, idx_dst, vbufs, stage0, sbuf, acc_sp,
                semv, sems0):
    c = lax.axis_index("c")
    s_ax = lax.axis_index("s")
    wid = c * 16 + s_ax
    ebase = wid * TE
    rbase = s_ax * RPT

    pltpu.sync_copy(src2_hbm.at[pl.ds(wid * NCH, NCH)], idx_src)
    pltpu.sync_copy(dst2_hbm.at[pl.ds(wid * NCH, NCH)], idx_dst)
    pltpu.sync_copy(s_hbm.at[pl.ds(ebase, TE)], sbuf)

    zf = jnp.zeros((16,), jnp.float32)

    def _zstage(r, _):
        for b in range(HH // 16):
            stage0[r, pl.ds(b * 16, 16)] = zf
        return 0
    lax.fori_loop(0, CH, _zstage, 0)
    for z in range(RPT // CH):
        pltpu.sync_copy(stage0, acc_sp.at[pl.ds(rbase + z * CH, CH)])
    plsc.subcore_barrier()

    def _issue(ci, b):
        ci = lax.rem(ci, NCH)
        pltpu.async_copy(vh_hbm.at[idx_src.at[ci]], vbufs.at[b], semv.at[b])

    def _drain(ci, b):
        ci = lax.rem(ci, NCH)
        pltpu.make_async_copy(vh_hbm.at[idx_src.at[ci]], vbufs.at[b],
                              semv.at[b]).wait()

    def _wait_sc(ci, b):
        pltpu.make_async_copy(stage0, acc_sp.at[idx_dst.at[ci]],
                              sems0).wait()

    def _slot(ci, b, first):
        vbuf = vbufs.at[b]
        _issue(ci + 1, 1 - b)
        _drain(ci, b)

        @pl.when(jnp.logical_not(first))
        def _():
            _wait_sc(ci, b)

        def _egroup(g2, _):
            wv = sbuf[pl.ds(ci * CH + g2 * 16, 16)]
            for j16 in range(16):
                j = g2 * 16 + j16
                w = wv[j16]
                for b2 in range(HH // 16):
                    stage0[j, pl.ds(b2 * 16, 16)] = \
                        vbuf[j, pl.ds(b2 * 16, 16)] * w
            return 0
        lax.fori_loop(0, CH // 16, _egroup, 0)
        pltpu.async_copy(stage0, acc_sp.at[idx_dst.at[ci]],
                         sems0, add=True)

    _issue(0, 0)

    def _pair(ci2, _):
        _slot(2 * ci2, 0, ci2 == 0)
        _slot(2 * ci2 + 1, 1, False)
        return 0
    lax.fori_loop(0, NCH // 2, _pair, 0)

    _drain(NCH, 0)
    _wait_sc(NCH - 1, 0)
    plsc.subcore_barrier()
    pltpu.sync_copy(acc_sp.at[pl.ds(rbase, RPT)],
                    acc_out.at[c, pl.ds(rbase, RPT)])


def _pass2(vh, s, src2, dst2):
    f = pl.kernel(
        _pass2_body,
        out_type=jax.ShapeDtypeStruct((2, NP, HH), jnp.float32),
        mesh=_MESH,
        compiler_params=_SC_PARAMS,
        scratch_types=[
            pltpu.VMEM((NCH, CH), jnp.int32),
            pltpu.VMEM((NCH, CH), jnp.int32),
            pltpu.VMEM((2, CH, HH), jnp.float32),
            pltpu.VMEM((CH, HH), jnp.float32),
            pltpu.VMEM((TE,), jnp.float32),
            pltpu.VMEM_SHARED((NP, HH), jnp.float32),
            pltpu.SemaphoreType.DMA((2,)),
            pltpu.SemaphoreType.DMA,
        ],
    )
    return f(vh, s, src2, dst2)


def _combine_body(acclo_ref, acchi_ref, agg_ref, skip_ref,
                  wet_ref, out_ref):
    a = agg_ref[0] + agg_ref[1]
    den = a[:, 16:17]
    inv = 1.0 / (den + 1e-16)
    lo = (acclo_ref[0] + acclo_ref[1]) * inv
    hi = (acchi_ref[0] + acchi_ref[1]) * inv
    corr = jnp.dot(a[:, 0:DE], wet_ref[...],
                   preferred_element_type=jnp.float32)
    out_ref[...] = jax.nn.relu(
        jnp.concatenate([lo, hi], axis=1) + corr + skip_ref[...])


def _combine(acclo, acchi, agg, skip, wet):
    return pl.pallas_call(
        _combine_body,
        grid=(NP // TCB,),
        in_specs=[
            pl.BlockSpec((2, TCB, HH), lambda i: (0, i, 0)),
            pl.BlockSpec((2, TCB, HH), lambda i: (0, i, 0)),
            pl.BlockSpec((2, TCB, 32), lambda i: (0, i, 0)),
            pl.BlockSpec((TCB, HID), lambda i: (i, 0)),
            pl.BlockSpec((DE, HID), lambda i: (0, 0)),
        ],
        out_specs=pl.BlockSpec((TCB, HID), lambda i: (i, 0)),
        out_shape=jax.ShapeDtypeStruct((NP, HID), jnp.float32),
    )(acclo, acchi, agg, skip, wet)


def _comb_proj_body(acclo_ref, acchi_ref, agg_ref, skip_ref, wet_ref,
                    w_ref, b_ref, we_ref,
                    q_ref, k_ref, vlo_ref, vhi_ref, skip2_ref, qw_ref):
    a = agg_ref[0] + agg_ref[1]
    den = a[:, 16:17]
    inv = 1.0 / (den + 1e-16)
    lo = (acclo_ref[0] + acclo_ref[1]) * inv
    hi = (acchi_ref[0] + acchi_ref[1]) * inv
    corr = jnp.dot(a[:, 0:DE], wet_ref[...],
                   preferred_element_type=jnp.float32)
    h = jax.nn.relu(
        jnp.concatenate([lo, hi], axis=1) + corr + skip_ref[...])
    big = jnp.dot(h, w_ref[...],
                  preferred_element_type=jnp.float32) + b_ref[...]
    q = big[:, 0:HID]
    q_ref[...] = q
    k_ref[...] = big[:, HID:2 * HID]
    vlo_ref[...] = big[:, 2 * HID:2 * HID + HH]
    vhi_ref[...] = big[:, 2 * HID + HH:3 * HID]
    skip2_ref[...] = big[:, 3 * HID:4 * HID]
    qw_ref[...] = jnp.dot(q, we_ref[...], preferred_element_type=jnp.float32)


def _comb_proj(acclo, acchi, agg, skip, wet, wcat, bcat, we):
    outs = [
        jax.ShapeDtypeStruct((NP, HID), jnp.float32),
        jax.ShapeDtypeStruct((NP, HID), jnp.float32),
        jax.ShapeDtypeStruct((NP, HH), jnp.float32),
        jax.ShapeDtypeStruct((NP, HH), jnp.float32),
        jax.ShapeDtypeStruct((NP, HID), jnp.float32),
        jax.ShapeDtypeStruct((NP, DE), jnp.float32),
    ]
    return pl.pallas_call(
        _comb_proj_body,
        grid=(NP // TCB,),
        in_specs=[
            pl.BlockSpec((2, TCB, HH), lambda i: (0, i, 0)),
            pl.BlockSpec((2, TCB, HH), lambda i: (0, i, 0)),
            pl.BlockSpec((2, TCB, 32), lambda i: (0, i, 0)),
            pl.BlockSpec((TCB, HID), lambda i: (i, 0)),
            pl.BlockSpec((DE, HID), lambda i: (0, 0)),
            pl.BlockSpec((D, 4 * HID), lambda i: (0, 0)),
            pl.BlockSpec((1, 4 * HID), lambda i: (0, 0)),
            pl.BlockSpec((D, DE), lambda i: (0, 0)),
        ],
        out_specs=[
            pl.BlockSpec((TCB, HID), lambda i: (i, 0)),
            pl.BlockSpec((TCB, HID), lambda i: (i, 0)),
            pl.BlockSpec((TCB, HH), lambda i: (i, 0)),
            pl.BlockSpec((TCB, HH), lambda i: (i, 0)),
            pl.BlockSpec((TCB, HID), lambda i: (i, 0)),
            pl.BlockSpec((TCB, DE), lambda i: (i, 0)),
        ],
        out_shape=outs,
    )(acclo, acchi, agg, skip, wet, wcat, bcat, we)

def _readout_body(h_ref, w1_ref, b1_ref, w2_ref, b2_ref, out_ref):
    pooled = jnp.sum(h_ref[0:N, :], axis=0, keepdims=True) * (1.0 / N)
    r = jax.nn.relu(jnp.dot(pooled, w1_ref[...],
                            preferred_element_type=jnp.float32) + b1_ref[...])
    out_ref[...] = jnp.dot(r, w2_ref[...],
                           preferred_element_type=jnp.float32) + b2_ref[...]


def _readout(h, w1t, b1, w2t, b2):
    return pl.pallas_call(
        _readout_body,
        out_shape=jax.ShapeDtypeStruct((1, 1), jnp.float32),
    )(h, w1t, b1, w2t, b2)


def kernel(x, edge_index, edge_attr, layer_params, readout_params):
    src = edge_index[0]
    dst = edge_index[1]
    src2 = jnp.pad(src, (0, EP - E)).reshape(EP // CH, CH)
    dst2 = jnp.pad(dst, (0, EP - E)).reshape(EP // CH, CH)
    ea_p = jnp.pad(edge_attr, ((0, EP - E), (0, 0)))
    h = jnp.pad(x, ((0, NP - N), (0, 0)))

    wcats, bcats = [], []
    for p in layer_params:
        wcats.append(jnp.concatenate(
            [p["Wq"].T, p["Wk"].T, p["Wv"].T, p["Ws"].T], axis=1))
        bcats.append(jnp.concatenate(
            [p["bq"], p["bk"], p["bv"], p["bs"]])[None, :])

    nl = len(layer_params)
    q, k, vlo, vhi, skip, qw = _proj(h, wcats[0], bcats[0],
                                     layer_params[0]["We"])
    for li, p in enumerate(layer_params):
        s, agg = _pass1(q, k, qw, ea_p, src2, dst2)
        acclo = _pass2(vlo, s, src2, dst2)
        acchi = _pass2(vhi, s, src2, dst2)
        if li + 1 < nl:
            pn = layer_params[li + 1]
            q, k, vlo, vhi, skip, qw = _comb_proj(
                acclo, acchi, agg, skip, p["We"].T,
                wcats[li + 1], bcats[li + 1], pn["We"])
        else:
            h = _combine(acclo, acchi, agg, skip, p["We"].T)

    rp = readout_params
    out = _readout(h, rp["W1"].T, rp["b1"][None, :], rp["W2"].T,
                   rp["b2"][None, :])
    return out[0]


# speedup vs baseline: 1.0112x; 1.0112x over previous
import jax
import jax.numpy as jnp
from jax import lax
from jax.experimental import pallas as pl
from jax.experimental.pallas import tpu as pltpu
from jax.experimental.pallas import tpu_sc as plsc

N = 10000
NP = 10240
E = 160000
EP = 163840
D = 256
DE = 16
HID = 256
HH = 128
SCALE = 1.0 / (HID ** 0.5)

NTILES = 32
TE = EP // NTILES
CH = 64
NCH = TE // CH
RPT = NP // 16
TCB = NP // 8

_MESH = plsc.VectorSubcoreMesh(core_axis_name="c", subcore_axis_name="s")
_SC_PARAMS = pltpu.CompilerParams(use_tc_tiling_on_sc=False,
                                  needs_layout_passes=False)


def _proj_body(h_ref, w_ref, b_ref, we_ref,
               q_ref, k_ref, vlo_ref, vhi_ref, skip_ref, qw_ref):
    big = jnp.dot(h_ref[...], w_ref[...],
                  preferred_element_type=jnp.float32) + b_ref[...]
    q = big[:, 0:HID]
    q_ref[...] = q
    k_ref[...] = big[:, HID:2 * HID]
    vlo_ref[...] = big[:, 2 * HID:2 * HID + HH]
    vhi_ref[...] = big[:, 2 * HID + HH:3 * HID]
    skip_ref[...] = big[:, 3 * HID:4 * HID]
    qw_ref[...] = jnp.dot(q, we_ref[...], preferred_element_type=jnp.float32)


def _proj(h, wcat, bcat, we):
    outs = [
        jax.ShapeDtypeStruct((NP, HID), jnp.float32),
        jax.ShapeDtypeStruct((NP, HID), jnp.float32),
        jax.ShapeDtypeStruct((NP, HH), jnp.float32),
        jax.ShapeDtypeStruct((NP, HH), jnp.float32),
        jax.ShapeDtypeStruct((NP, HID), jnp.float32),
        jax.ShapeDtypeStruct((NP, DE), jnp.float32),
    ]
    return pl.pallas_call(
        _proj_body,
        grid=(NP // TCB,),
        in_specs=[
            pl.BlockSpec((TCB, D), lambda i: (i, 0)),
            pl.BlockSpec((D, 4 * HID), lambda i: (0, 0)),
            pl.BlockSpec((1, 4 * HID), lambda i: (0, 0)),
            pl.BlockSpec((D, DE), lambda i: (0, 0)),
        ],
        out_specs=[
            pl.BlockSpec((TCB, HID), lambda i: (i, 0)),
            pl.BlockSpec((TCB, HID), lambda i: (i, 0)),
            pl.BlockSpec((TCB, HH), lambda i: (i, 0)),
            pl.BlockSpec((TCB, HH), lambda i: (i, 0)),
            pl.BlockSpec((TCB, HID), lambda i: (i, 0)),
            pl.BlockSpec((TCB, DE), lambda i: (i, 0)),
        ],
        out_shape=outs,
    )(h, wcat, bcat, we)


def _pass1_body(q_hbm, k_hbm, qw_hbm, ea_hbm, src2_hbm, dst2_hbm,
                s_out, agg_out,
                idx_src, idx_dst, qbufs, kbufs, qwbufs, eabufs,
                stage0, sbuf, agg_sp, semg, sems0):
    c = lax.axis_index("c")
    s_ax = lax.axis_index("s")
    wid = c * 16 + s_ax
    ebase = wid * TE
    rbase = s_ax * RPT

    pltpu.sync_copy(src2_hbm.at[pl.ds(wid * NCH, NCH)], idx_src)
    pltpu.sync_copy(dst2_hbm.at[pl.ds(wid * NCH, NCH)], idx_dst)

    zf = jnp.zeros((16,), jnp.float32)

    def _zstage(r, _):
        stage0[r, pl.ds(0, 16)] = zf
        stage0[r, pl.ds(16, 16)] = zf
        return 0
    lax.fori_loop(0, CH, _zstage, 0)

    for z in range(RPT // CH):
        pltpu.sync_copy(stage0, agg_sp.at[pl.ds(rbase + z * CH, CH)])
    plsc.subcore_barrier()

    lanes = lax.iota(jnp.int32, 16)

    def _issue(ci, b):
        ci = lax.rem(ci, NCH)
        pltpu.async_copy(q_hbm.at[idx_dst.at[ci]], qbufs.at[b], semg.at[b])
        pltpu.async_copy(k_hbm.at[idx_src.at[ci]], kbufs.at[b], semg.at[b])
        pltpu.async_copy(qw_hbm.at[idx_dst.at[ci]], qwbufs.at[b], semg.at[b])
        pltpu.async_copy(ea_hbm.at[pl.ds(ebase + ci * CH, CH)],
                         eabufs.at[b], semg.at[b])

    def _drain(ci, b):
        ci = lax.rem(ci, NCH)
        pltpu.make_async_copy(q_hbm.at[idx_dst.at[ci]], qbufs.at[b],
                              semg.at[b]).wait()
        pltpu.make_async_copy(k_hbm.at[idx_src.at[ci]], kbufs.at[b],
                              semg.at[b]).wait()
        pltpu.make_async_copy(qw_hbm.at[idx_dst.at[ci]], qwbufs.at[b],
                              semg.at[b]).wait()
        pltpu.make_async_copy(ea_hbm.at[pl.ds(ebase + ci * CH, CH)],
                              eabufs.at[b], semg.at[b]).wait()

    def _wait_sc(ci, b):
        pltpu.make_async_copy(stage0, agg_sp.at[idx_dst.at[ci]],
                              sems0).wait()

    def _slot(ci, b, first):
        qbuf, kbuf = qbufs.at[b], kbufs.at[b]
        qwbuf, eabuf = qwbufs.at[b], eabufs.at[b]
        _issue(ci + 1, 1 - b)
        _drain(ci, b)
        for g in range(CH // 16):
            ids = g * 16 + lanes
            gid = ebase + ci * CH + ids
            mask = gid < E

            def _dot(f, acc):
                col = jnp.full((16,), f, jnp.int32)
                qv = plsc.load_gather(qbuf, [ids, col])
                kv = plsc.load_gather(kbuf, [ids, col])
                return acc + qv * kv
            acc = lax.fori_loop(0, HID, _dot, jnp.zeros((16,), jnp.float32),
                                unroll=8)
            for f in range(DE):
                col = jnp.full((16,), f, jnp.int32)
                acc = acc + (plsc.load_gather(qwbuf, [ids, col])
                             * plsc.load_gather(eabuf, [ids, col]))
            sv = jnp.where(mask, jnp.exp(acc * SCALE), 0.0)
            sbuf[pl.ds(ci * CH + g * 16, 16)] = sv

        @pl.when(jnp.logical_not(first))
        def _():
            _wait_sc(ci, b)

        def _egroup(g2, _):
            wv = sbuf[pl.ds(ci * CH + g2 * 16, 16)]
            for j16 in range(16):
                j = g2 * 16 + j16
                w = wv[j16]
                stage0[j, pl.ds(0, 16)] = eabuf[j, :] * w
                stage0[j, pl.ds(16, 16)] = jnp.full((16,), w, jnp.float32)
            return 0
        lax.fori_loop(0, CH // 16, _egroup, 0)

        pltpu.async_copy(stage0, agg_sp.at[idx_dst.at[ci]],
                         sems0, add=True)

    _issue(0, 0)

    def _pair(ci2, _):
        _slot(2 * ci2, 0, ci2 == 0)
        _slot(2 * ci2 + 1, 1, False)
        return 0
    lax.fori_loop(0, NCH // 2, _pair, 0)

    _drain(NCH, 0)
    _wait_sc(NCH - 1, 0)
    pltpu.sync_copy(sbuf, s_out.at[pl.ds(ebase, TE)])

    plsc.subcore_barrier()
    pltpu.sync_copy(agg_sp.at[pl.ds(rbase, RPT)],
                    agg_out.at[c, pl.ds(rbase, RPT)])


def _pass1(q, k, qw, ea_p, src2, dst2):
    f = pl.kernel(
        _pass1_body,
        out_type=[
            jax.ShapeDtypeStruct((EP,), jnp.float32),
            jax.ShapeDtypeStruct((2, NP, 32), jnp.float32),
        ],
        mesh=_MESH,
        compiler_params=_SC_PARAMS,
        scratch_types=[
            pltpu.VMEM((NCH, CH), jnp.int32),
            pltpu.VMEM((NCH, CH), jnp.int32),
            pltpu.VMEM((2, CH, HID), jnp.float32),
            pltpu.VMEM((2, CH, HID), jnp.float32),
            pltpu.VMEM((2, CH, DE), jnp.float32),
            pltpu.VMEM((2, CH, DE), jnp.float32),
            pltpu.VMEM((CH, 32), jnp.float32),
            pltpu.VMEM((TE,), jnp.float32),
            pltpu.VMEM_SHARED((NP, 32), jnp.float32),
            pltpu.SemaphoreType.DMA((2,)),
            pltpu.SemaphoreType.DMA,
        ],
    )
    return f(q, k, qw, ea_p, src2, dst2)


def _pass2_body(vh_hbm, s_hbm, src2_hbm, dst32_hbm,
                acc_out,
                idx_src, idx_dst32, vbufs, stage0, sbuf, acc_sp,
                semv, sems0, sems1):
    c = lax.axis_index("c")
    s_ax = lax.axis_index("s")
    wid = c * 16 + s_ax
    ebase = wid * TE
    rbase = s_ax * RPT

    pltpu.sync_copy(src2_hbm.at[pl.ds(wid * NCH, NCH)], idx_src)
    pltpu.sync_copy(dst32_hbm.at[pl.ds(wid * NCH * 2, NCH * 2)], idx_dst32)
    pltpu.sync_copy(s_hbm.at[pl.ds(ebase, TE)], sbuf)

    zf = jnp.zeros((16,), jnp.float32)

    def _zstage(r, _):
        for b in range(HH // 16):
            stage0[r, pl.ds(b * 16, 16)] = zf
        return 0
    lax.fori_loop(0, CH, _zstage, 0)
    for z in range(RPT // CH):
        pltpu.sync_copy(stage0, acc_sp.at[pl.ds(rbase + z * CH, CH)])
    plsc.subcore_barrier()

    def _issue(ci, b):
        ci = lax.rem(ci, NCH)
        pltpu.async_copy(vh_hbm.at[idx_src.at[ci]], vbufs.at[b], semv.at[b])

    def _drain(ci, b):
        ci = lax.rem(ci, NCH)
        pltpu.make_async_copy(vh_hbm.at[idx_src.at[ci]], vbufs.at[b],
                              semv.at[b]).wait()

    def _wait_sc(ci, b):
        pltpu.make_async_copy(stage0.at[pl.ds(0, CH // 2)],
                              acc_sp.at[idx_dst32.at[2 * ci]],
                              sems0).wait()
        pltpu.make_async_copy(stage0.at[pl.ds(CH // 2, CH // 2)],
                              acc_sp.at[idx_dst32.at[2 * ci + 1]],
                              sems1).wait()

    def _slot(ci, b, first):
        vbuf = vbufs.at[b]
        _issue(ci + 1, 1 - b)
        _drain(ci, b)

        @pl.when(jnp.logical_not(first))
        def _():
            _wait_sc(ci, b)

        def _egroup(g2, _):
            wv = sbuf[pl.ds(ci * CH + g2 * 16, 16)]
            for j16 in range(16):
                j = g2 * 16 + j16
                w = wv[j16]
                for b2 in range(HH // 16):
                    stage0[j, pl.ds(b2 * 16, 16)] = \
                        vbuf[j, pl.ds(b2 * 16, 16)] * w
            return 0
        lax.fori_loop(0, CH // 16, _egroup, 0)
        pltpu.async_copy(stage0.at[pl.ds(0, CH // 2)],
                         acc_sp.at[idx_dst32.at[2 * ci]],
                         sems0, add=True)
        pltpu.async_copy(stage0.at[pl.ds(CH // 2, CH // 2)],
                         acc_sp.at[idx_dst32.at[2 * ci + 1]],
                         sems1, add=True)

    _issue(0, 0)

    def _pair(ci2, _):
        _slot(2 * ci2, 0, ci2 == 0)
        _slot(2 * ci2 + 1, 1, False)
        return 0
    lax.fori_loop(0, NCH // 2, _pair, 0)

    _drain(NCH, 0)
    _wait_sc(NCH - 1, 0)
    plsc.subcore_barrier()
    pltpu.sync_copy(acc_sp.at[pl.ds(rbase, RPT)],
                    acc_out.at[c, pl.ds(rbase, RPT)])


def _pass2(vh, s, src2, dst32):
    f = pl.kernel(
        _pass2_body,
        out_type=jax.ShapeDtypeStruct((2, NP, HH), jnp.float32),
        mesh=_MESH,
        compiler_params=_SC_PARAMS,
        scratch_types=[
            pltpu.VMEM((NCH, CH), jnp.int32),
            pltpu.VMEM((NCH * 2, CH // 2), jnp.int32),
            pltpu.VMEM((2, CH, HH), jnp.float32),
            pltpu.VMEM((CH, HH), jnp.float32),
            pltpu.VMEM((TE,), jnp.float32),
            pltpu.VMEM_SHARED((NP, HH), jnp.float32),
            pltpu.SemaphoreType.DMA((2,)),
            pltpu.SemaphoreType.DMA,
            pltpu.SemaphoreType.DMA,
        ],
    )
    return f(vh, s, src2, dst32)


def _combine_body(acclo_ref, acchi_ref, agg_ref, skip_ref,
                  wet_ref, out_ref):
    a = agg_ref[0] + agg_ref[1]
    den = a[:, 16:17]
    inv = 1.0 / (den + 1e-16)
    lo = (acclo_ref[0] + acclo_ref[1]) * inv
    hi = (acchi_ref[0] + acchi_ref[1]) * inv
    corr = jnp.dot(a[:, 0:DE], wet_ref[...],
                   preferred_element_type=jnp.float32)
    out_ref[...] = jax.nn.relu(
        jnp.concatenate([lo, hi], axis=1) + corr + skip_ref[...])


def _combine(acclo, acchi, agg, skip, wet):
    return pl.pallas_call(
        _combine_body,
        grid=(NP // TCB,),
        in_specs=[
            pl.BlockSpec((2, TCB, HH), lambda i: (0, i, 0)),
            pl.BlockSpec((2, TCB, HH), lambda i: (0, i, 0)),
            pl.BlockSpec((2, TCB, 32), lambda i: (0, i, 0)),
            pl.BlockSpec((TCB, HID), lambda i: (i, 0)),
            pl.BlockSpec((DE, HID), lambda i: (0, 0)),
        ],
        out_specs=pl.BlockSpec((TCB, HID), lambda i: (i, 0)),
        out_shape=jax.ShapeDtypeStruct((NP, HID), jnp.float32),
    )(acclo, acchi, agg, skip, wet)


def _readout_body(h_ref, w1_ref, b1_ref, w2_ref, b2_ref, out_ref):
    pooled = jnp.sum(h_ref[0:N, :], axis=0, keepdims=True) * (1.0 / N)
    r = jax.nn.relu(jnp.dot(pooled, w1_ref[...],
                            preferred_element_type=jnp.float32) + b1_ref[...])
    out_ref[...] = jnp.dot(r, w2_ref[...],
                           preferred_element_type=jnp.float32) + b2_ref[...]


def _readout(h, w1t, b1, w2t, b2):
    return pl.pallas_call(
        _readout_body,
        out_shape=jax.ShapeDtypeStruct((1, 1), jnp.float32),
    )(h, w1t, b1, w2t, b2)


def kernel(x, edge_index, edge_attr, layer_params, readout_params):
    src = edge_index[0]
    dst = edge_index[1]
    src2 = jnp.pad(src, (0, EP - E)).reshape(EP // CH, CH)
    dstp = jnp.pad(dst, (0, EP - E))
    dst2 = dstp.reshape(EP // CH, CH)
    dst32 = dstp.reshape(EP // (CH // 2), CH // 2)
    ea_p = jnp.pad(edge_attr, ((0, EP - E), (0, 0)))
    h = jnp.pad(x, ((0, NP - N), (0, 0)))

    for p in layer_params:
        wcat = jnp.concatenate(
            [p["Wq"].T, p["Wk"].T, p["Wv"].T, p["Ws"].T], axis=1)
        bcat = jnp.concatenate(
            [p["bq"], p["bk"], p["bv"], p["bs"]])[None, :]
        q, k, vlo, vhi, skip, qw = _proj(h, wcat, bcat, p["We"])
        s, agg = _pass1(q, k, qw, ea_p, src2, dst2)
        acclo = _pass2(vlo, s, src2, dst32)
        acchi = _pass2(vhi, s, src2, dst32)
        h = _combine(acclo, acchi, agg, skip, p["We"].T)

    rp = readout_params
    out = _readout(h, rp["W1"].T, rp["b1"][None, :], rp["W2"].T,
                   rp["b2"][None, :])
    return out[0]


# speedup vs baseline: 1.0113x; 1.0001x over previous
import jax
import jax.numpy as jnp
from jax import lax
from jax.experimental import pallas as pl
from jax.experimental.pallas import tpu as pltpu
from jax.experimental.pallas import tpu_sc as plsc

N = 10000
NP = 10240
E = 160000
EP = 163840
D = 256
DE = 16
HID = 256
HH = 128
SCALE = 1.0 / (HID ** 0.5)

NTILES = 32
TE = EP // NTILES
CH = 64
NCH = TE // CH
RPT = NP // 16
TCB = NP // 8

_MESH = plsc.VectorSubcoreMesh(core_axis_name="c", subcore_axis_name="s")
_SC_PARAMS = pltpu.CompilerParams(use_tc_tiling_on_sc=False,
                                  needs_layout_passes=False)


def _proj_body(h_ref, w_ref, b_ref, we_ref,
               q_ref, k_ref, vlo_ref, vhi_ref, skip_ref, qw_ref):
    big = jnp.dot(h_ref[...], w_ref[...],
                  preferred_element_type=jnp.float32) + b_ref[...]
    q = big[:, 0:HID]
    q_ref[...] = q
    k_ref[...] = big[:, HID:2 * HID]
    vlo_ref[...] = big[:, 2 * HID:2 * HID + HH]
    vhi_ref[...] = big[:, 2 * HID + HH:3 * HID]
    skip_ref[...] = big[:, 3 * HID:4 * HID]
    qw_ref[...] = jnp.dot(q, we_ref[...], preferred_element_type=jnp.float32)


def _proj(h, wcat, bcat, we):
    outs = [
        jax.ShapeDtypeStruct((NP, HID), jnp.float32),
        jax.ShapeDtypeStruct((NP, HID), jnp.float32),
        jax.ShapeDtypeStruct((NP, HH), jnp.float32),
        jax.ShapeDtypeStruct((NP, HH), jnp.float32),
        jax.ShapeDtypeStruct((NP, HID), jnp.float32),
        jax.ShapeDtypeStruct((NP, DE), jnp.float32),
    ]
    return pl.pallas_call(
        _proj_body,
        grid=(NP // TCB,),
        in_specs=[
            pl.BlockSpec((TCB, D), lambda i: (i, 0)),
            pl.BlockSpec((D, 4 * HID), lambda i: (0, 0)),
            pl.BlockSpec((1, 4 * HID), lambda i: (0, 0)),
            pl.BlockSpec((D, DE), lambda i: (0, 0)),
        ],
        out_specs=[
            pl.BlockSpec((TCB, HID), lambda i: (i, 0)),
            pl.BlockSpec((TCB, HID), lambda i: (i, 0)),
            pl.BlockSpec((TCB, HH), lambda i: (i, 0)),
            pl.BlockSpec((TCB, HH), lambda i: (i, 0)),
            pl.BlockSpec((TCB, HID), lambda i: (i, 0)),
            pl.BlockSpec((TCB, DE), lambda i: (i, 0)),
        ],
        out_shape=outs,
    )(h, wcat, bcat, we)


def _pass1_body(q_hbm, k_hbm, qw_hbm, ea_hbm, src2_hbm, dst2_hbm,
                s_out, agg_out,
                idx_src, idx_dst, qbufs, kbufs, qwbufs, eabufs,
                stage0, sbuf, agg_sp, semg, sems0):
    c = lax.axis_index("c")
    s_ax = lax.axis_index("s")
    wid = c * 16 + s_ax
    ebase = wid * TE
    rbase = s_ax * RPT

    pltpu.sync_copy(src2_hbm.at[pl.ds(wid * NCH, NCH)], idx_src)
    pltpu.sync_copy(dst2_hbm.at[pl.ds(wid * NCH, NCH)], idx_dst)

    zf = jnp.zeros((16,), jnp.float32)

    def _zstage(r, _):
        stage0[r, pl.ds(0, 16)] = zf
        stage0[r, pl.ds(16, 16)] = zf
        return 0
    lax.fori_loop(0, CH, _zstage, 0)

    for z in range(RPT // CH):
        pltpu.sync_copy(stage0, agg_sp.at[pl.ds(rbase + z * CH, CH)])
    plsc.subcore_barrier()

    lanes = lax.iota(jnp.int32, 16)

    def _issue(ci, b):
        ci = lax.rem(ci, NCH)
        pltpu.async_copy(q_hbm.at[idx_dst.at[ci]], qbufs.at[b], semg.at[b])
        pltpu.async_copy(k_hbm.at[idx_src.at[ci]], kbufs.at[b], semg.at[b])
        pltpu.async_copy(qw_hbm.at[idx_dst.at[ci]], qwbufs.at[b], semg.at[b])
        pltpu.async_copy(ea_hbm.at[pl.ds(ebase + ci * CH, CH)],
                         eabufs.at[b], semg.at[b])

    def _drain(ci, b):
        ci = lax.rem(ci, NCH)
        pltpu.make_async_copy(q_hbm.at[idx_dst.at[ci]], qbufs.at[b],
                              semg.at[b]).wait()
        pltpu.make_async_copy(k_hbm.at[idx_src.at[ci]], kbufs.at[b],
                              semg.at[b]).wait()
        pltpu.make_async_copy(qw_hbm.at[idx_dst.at[ci]], qwbufs.at[b],
                              semg.at[b]).wait()
        pltpu.make_async_copy(ea_hbm.at[pl.ds(ebase + ci * CH, CH)],
                              eabufs.at[b], semg.at[b]).wait()

    def _wait_sc(ci, b):
        pltpu.make_async_copy(stage0, agg_sp.at[idx_dst.at[ci]],
                              sems0).wait()

    def _slot(ci, b, first):
        qbuf, kbuf = qbufs.at[b], kbufs.at[b]
        qwbuf, eabuf = qwbufs.at[b], eabufs.at[b]
        _issue(ci + 1, 1 - b)
        _drain(ci, b)
        for g in range(CH // 16):
            ids = g * 16 + lanes
            gid = ebase + ci * CH + ids
            mask = gid < E

            def _dot(f, acc):
                col = jnp.full((16,), f, jnp.int32)
                qv = plsc.load_gather(qbuf, [ids, col])
                kv = plsc.load_gather(kbuf, [ids, col])
                return acc + qv * kv
            acc = lax.fori_loop(0, HID, _dot, jnp.zeros((16,), jnp.float32),
                                unroll=8)
            for f in range(DE):
                col = jnp.full((16,), f, jnp.int32)
                acc = acc + (plsc.load_gather(qwbuf, [ids, col])
                             * plsc.load_gather(eabuf, [ids, col]))
            sv = jnp.where(mask, jnp.exp(acc * SCALE), 0.0)
            sbuf[pl.ds(ci * CH + g * 16, 16)] = sv

        @pl.when(jnp.logical_not(first))
        def _():
            _wait_sc(ci, b)

        def _egroup(g2, _):
            wv = sbuf[pl.ds(ci * CH + g2 * 16, 16)]
            for j16 in range(16):
                j = g2 * 16 + j16
                w = wv[j16]
                stage0[j, pl.ds(0, 16)] = eabuf[j, :] * w
                stage0[j, pl.ds(16, 16)] = jnp.full((16,), w, jnp.float32)
            return 0
        lax.fori_loop(0, CH // 16, _egroup, 0)

        pltpu.async_copy(stage0, agg_sp.at[idx_dst.at[ci]],
                         sems0, add=True)

    _issue(0, 0)

    def _pair(ci2, _):
        _slot(2 * ci2, 0, ci2 == 0)
        _slot(2 * ci2 + 1, 1, False)
        return 0
    lax.fori_loop(0, NCH // 2, _pair, 0)

    _drain(NCH, 0)
    _wait_sc(NCH - 1, 0)
    pltpu.sync_copy(sbuf, s_out.at[pl.ds(ebase, TE)])

    plsc.subcore_barrier()
    pltpu.sync_copy(agg_sp.at[pl.ds(rbase, RPT)],
                    agg_out.at[c, pl.ds(rbase, RPT)])


def _pass1(q, k, qw, ea_p, src2, dst2):
    f = pl.kernel(
        _pass1_body,
        out_type=[
            jax.ShapeDtypeStruct((EP,), jnp.float32),
            jax.ShapeDtypeStruct((2, NP, 32), jnp.float32),
        ],
        mesh=_MESH,
        compiler_params=_SC_PARAMS,
        scratch_types=[
            pltpu.VMEM((NCH, CH), jnp.int32),
            pltpu.VMEM((NCH, CH), jnp.int32),
            pltpu.VMEM((2, CH, HID), jnp.float32),
            pltpu.VMEM((2, CH, HID), jnp.float32),
            pltpu.VMEM((2, CH, DE), jnp.float32),
            pltpu.VMEM((2, CH, DE), jnp.float32),
            pltpu.VMEM((CH, 32), jnp.float32),
            pltpu.VMEM((TE,), jnp.float32),
            pltpu.VMEM_SHARED((NP, 32), jnp.float32),
            pltpu.SemaphoreType.DMA((2,)),
            pltpu.SemaphoreType.DMA,
        ],
    )
    return f(q, k, qw, ea_p, src2, dst2)


def _pass2_body(vh_hbm, s_hbm, src2_hbm, dst2_hbm,
                acc_out,
                idx_src, idx_dst, vbufs, stage0, sbuf, acc_sp,
                semv, sems0):
    c = lax.axis_index("c")
    s_ax = lax.axis_index("s")
    wid = c * 16 + s_ax
    ebase = wid * TE
    rbase = s_ax * RPT

    pltpu.sync_copy(src2_hbm.at[pl.ds(wid * NCH, NCH)], idx_src)
    pltpu.sync_copy(dst2_hbm.at[pl.ds(wid * NCH, NCH)], idx_dst)
    pltpu.sync_copy(s_hbm.at[pl.ds(ebase, TE)], sbuf)

    zf = jnp.zeros((16,), jnp.float32)

    def _zstage(r, _):
        for b in range(HH // 16):
            stage0[r, pl.ds(b * 16, 16)] = zf
        return 0
    lax.fori_loop(0, CH, _zstage, 0)
    for z in range(RPT // CH):
        pltpu.sync_copy(stage0, acc_sp.at[pl.ds(rbase + z * CH, CH)])
    plsc.subcore_barrier()

    def _issue(ci, b):
        ci = lax.rem(ci, NCH)
        pltpu.async_copy(vh_hbm.at[idx_src.at[ci]], vbufs.at[b], semv.at[b])

    def _drain(ci, b):
        ci = lax.rem(ci, NCH)
        pltpu.make_async_copy(vh_hbm.at[idx_src.at[ci]], vbufs.at[b],
                              semv.at[b]).wait()

    def _wait_sc(ci, b):
        pltpu.make_async_copy(stage0, acc_sp.at[idx_dst.at[ci]],
                              sems0).wait()

    def _slot(ci, b, first):
        vbuf = vbufs.at[b]
        _issue(ci + 1, 1 - b)
        _drain(ci, b)

        @pl.when(jnp.logical_not(first))
        def _():
            _wait_sc(ci, b)

        def _egroup(g2, _):
            wv = sbuf[pl.ds(ci * CH + g2 * 16, 16)]
            for j16 in range(16):
                j = g2 * 16 + j16
                w = wv[j16]
                for b2 in range(HH // 16):
                    stage0[j, pl.ds(b2 * 16, 16)] = \
                        vbuf[j, pl.ds(b2 * 16, 16)] * w
            return 0
        lax.fori_loop(0, CH // 16, _egroup, 0)
        pltpu.async_copy(stage0, acc_sp.at[idx_dst.at[ci]],
                         sems0, add=True)

    _issue(0, 0)

    def _pair(ci2, _):
        _slot(2 * ci2, 0, ci2 == 0)
        _slot(2 * ci2 + 1, 1, False)
        return 0
    lax.fori_loop(0, NCH // 2, _pair, 0)

    _drain(NCH, 0)
    _wait_sc(NCH - 1, 0)
    plsc.subcore_barrier()
    pltpu.sync_copy(acc_sp.at[pl.ds(rbase, RPT)],
                    acc_out.at[c, pl.ds(rbase, RPT)])


def _pass2(vh, s, src2, dst2):
    f = pl.kernel(
        _pass2_body,
        out_type=jax.ShapeDtypeStruct((2, NP, HH), jnp.float32),
        mesh=_MESH,
        compiler_params=_SC_PARAMS,
        scratch_types=[
            pltpu.VMEM((NCH, CH), jnp.int32),
            pltpu.VMEM((NCH, CH), jnp.int32),
            pltpu.VMEM((2, CH, HH), jnp.float32),
            pltpu.VMEM((CH, HH), jnp.float32),
            pltpu.VMEM((TE,), jnp.float32),
            pltpu.VMEM_SHARED((NP, HH), jnp.float32),
            pltpu.SemaphoreType.DMA((2,)),
            pltpu.SemaphoreType.DMA,
        ],
    )
    return f(vh, s, src2, dst2)


def _combine_body(acclo_ref, acchi_ref, agg_ref, skip_ref,
                  wet_ref, out_ref):
    a = agg_ref[0] + agg_ref[1]
    den = a[:, 16:17]
    inv = 1.0 / (den + 1e-16)
    lo = (acclo_ref[0] + acclo_ref[1]) * inv
    hi = (acchi_ref[0] + acchi_ref[1]) * inv
    corr = jnp.dot(a[:, 0:DE], wet_ref[...],
                   preferred_element_type=jnp.float32)
    out_ref[...] = jax.nn.relu(
        jnp.concatenate([lo, hi], axis=1) + corr + skip_ref[...])


def _combine(acclo, acchi, agg, skip, wet):
    return pl.pallas_call(
        _combine_body,
        grid=(NP // TCB,),
        in_specs=[
            pl.BlockSpec((2, TCB, HH), lambda i: (0, i, 0)),
            pl.BlockSpec((2, TCB, HH), lambda i: (0, i, 0)),
            pl.BlockSpec((2, TCB, 32), lambda i: (0, i, 0)),
            pl.BlockSpec((TCB, HID), lambda i: (i, 0)),
            pl.BlockSpec((DE, HID), lambda i: (0, 0)),
        ],
        out_specs=pl.BlockSpec((TCB, HID), lambda i: (i, 0)),
        out_shape=jax.ShapeDtypeStruct((NP, HID), jnp.float32),
    )(acclo, acchi, agg, skip, wet)


def _readout_body(h_ref, w1_ref, b1_ref, w2_ref, b2_ref, out_ref):
    pooled = jnp.sum(h_ref[0:N, :], axis=0, keepdims=True) * (1.0 / N)
    r = jax.nn.relu(jnp.dot(pooled, w1_ref[...],
                            preferred_element_type=jnp.float32) + b1_ref[...])
    out_ref[...] = jnp.dot(r, w2_ref[...],
                           preferred_element_type=jnp.float32) + b2_ref[...]


def _readout(h, w1t, b1, w2t, b2):
    return pl.pallas_call(
        _readout_body,
        out_shape=jax.ShapeDtypeStruct((1, 1), jnp.float32),
    )(h, w1t, b1, w2t, b2)


def kernel(x, edge_index, edge_attr, layer_params, readout_params):
    src = edge_index[0]
    dst = edge_index[1]
    src2 = jnp.pad(src, (0, EP - E)).reshape(EP // CH, CH)
    dst2 = jnp.pad(dst, (0, EP - E)).reshape(EP // CH, CH)
    ea_p = jnp.pad(edge_attr, ((0, EP - E), (0, 0)))
    h = jnp.pad(x, ((0, NP - N), (0, 0)))

    for p in layer_params:
        wcat = jnp.concatenate(
            [p["Wq"].T, p["Wk"].T, p["Wv"].T, p["Ws"].T], axis=1)
        bcat = jnp.concatenate(
            [p["bq"], p["bk"], p["bv"], p["bs"]])[None, :]
        q, k, vlo, vhi, skip, qw = _proj(h, wcat, bcat, p["We"])
        s, agg = _pass1(q, k, qw, ea_p, src2, dst2)
        acclo = _pass2(vlo, s, src2, dst2)
        acchi = _pass2(vhi, s, src2, dst2)
        h = _combine(acclo, acchi, agg, skip, p["We"].T)

    rp = readout_params
    out = _readout(h, rp["W1"].T, rp["b1"][None, :], rp["W2"].T,
                   rp["b2"][None, :])
    return out[0]
